# baseline (speedup 1.0000x reference)
import functools

import jax
import jax.numpy as jnp
from jax import lax
from jax.experimental import pallas as pl
from jax.experimental.pallas import tpu as pltpu
from jax.experimental.pallas import tpu_sc as plsc

B = 512
NU = 1000
NI = 1000
D = 64
FU = 4
FI = 4
IN = D + FU + D + FI
H = D

SC_NC = 2
SC_NS = 16
SC_NW = SC_NC * SC_NS
B_PER_W = B // SC_NW


def _sc_gather(u_flat, i_flat, u_idx, i_idx):
    mesh = plsc.VectorSubcoreMesh(core_axis_name="c", subcore_axis_name="s")

    @functools.partial(
        pl.kernel,
        mesh=mesh,
        out_type=(jax.ShapeDtypeStruct((B, 2 * D), jnp.float32),
                  jax.ShapeDtypeStruct((B, 2 * D), jnp.float32)),
        scratch_types=[
            pltpu.VMEM((B_PER_W,), jnp.int32),
            pltpu.VMEM((B_PER_W, 2 * D), jnp.float32),
            pltpu.SemaphoreType.DMA,
        ],
    )
    def gather_kernel(u_tab, i_tab, u_ix, i_ix, u_out, i_out, idx_v, rows_v, sem):
        wid = lax.axis_index("s") * SC_NC + lax.axis_index("c")
        base = wid * B_PER_W
        pltpu.sync_copy(u_ix.at[pl.ds(base, B_PER_W)], idx_v)
        pltpu.async_copy(u_tab.at[idx_v], rows_v, sem).wait()
        pltpu.sync_copy(rows_v, u_out.at[pl.ds(base, B_PER_W)])
        pltpu.sync_copy(i_ix.at[pl.ds(base, B_PER_W)], idx_v)
        pltpu.async_copy(i_tab.at[idx_v], rows_v, sem).wait()
        pltpu.sync_copy(rows_v, i_out.at[pl.ds(base, B_PER_W)])

    return gather_kernel(u_flat, i_flat, u_idx, i_idx)


def _gru_body(ue2, ie2, up, ip_, uf, if_, wih_u, bih_u, bhh_u,
              wih_i, bih_i, bhh_i, uo, io):
    ue = jnp.where(up[...] == 1, ue2[:, D:], ue2[:, :D])
    ie = jnp.where(ip_[...] == 1, ie2[:, D:], ie2[:, :D])
    for emb, feat, emb2, feat2, wih, bih, bhh, out in (
        (ue, uf, ie, if_, wih_u, bih_u, bhh_u, uo),
        (ie, if_, ue, uf, wih_i, bih_i, bhh_i, io),
    ):
        x = jnp.concatenate(
            [emb, feat[...], emb2, feat2[...]], axis=1)
        gi = lax.dot_general(x, wih[...], (((1,), (1,)), ((), ())),
                             preferred_element_type=jnp.float32)
        gi = gi + bih[...]
        bhh_v = bhh[...]
        r = jax.nn.sigmoid(gi[:, :H] + bhh_v[:, :H])
        z = jax.nn.sigmoid(gi[:, H:2 * H] + bhh_v[:, H:2 * H])
        n = jnp.tanh(gi[:, 2 * H:] + r * bhh_v[:, 2 * H:])
        h = (1.0 - z) * n
        nrm = jnp.sqrt(jnp.sum(h * h, axis=1, keepdims=True))
        out[...] = h / jnp.maximum(nrm, 1e-12)


def _tc_gru(ue2, ie2, up, ip_, uf, if_, wih_u, bih_u, bhh_u,
            wih_i, bih_i, bhh_i):
    return pl.pallas_call(
        _gru_body,
        out_shape=(jax.ShapeDtypeStruct((B, H), jnp.float32),
                   jax.ShapeDtypeStruct((B, H), jnp.float32)),
    )(ue2, ie2, up, ip_, uf, if_, wih_u, bih_u, bhh_u, wih_i, bih_i, bhh_i)


SCATTER_BB = 8


def _scatter_body(mem_ref, emb_ref, ids_ref, out_ref):
    out_ref[...] = mem_ref[...]
    b0 = pl.program_id(0) * SCATTER_BB
    for i in range(SCATTER_BB):
        idx = ids_ref[b0 + i]
        out_ref[i, pl.ds(idx, 1), :] = emb_ref[pl.ds(i, 1), :]


def _tc_copy_scatter(mem, new_emb, ids, n_rows):
    grid = (B // SCATTER_BB,)
    return pl.pallas_call(
        _scatter_body,
        grid=grid,
        in_specs=[
            pl.BlockSpec((SCATTER_BB, n_rows, D), lambda b: (b, 0, 0)),
            pl.BlockSpec((SCATTER_BB, D), lambda b: (b, 0)),
            pl.BlockSpec(memory_space=pltpu.SMEM),
        ],
        out_specs=pl.BlockSpec((SCATTER_BB, n_rows, D), lambda b: (b, 0, 0)),
        out_shape=jax.ShapeDtypeStruct((B, n_rows, D), jnp.float32),
    )(mem, new_emb, ids)


def kernel(user_ids, item_ids, user_features, item_features,
           user_memory, item_memory,
           w_ih_u, w_hh_u, b_ih_u, b_hh_u,
           w_ih_i, w_hh_i, b_ih_i, b_hh_i):
    user_ids = user_ids.astype(jnp.int32)
    item_ids = item_ids.astype(jnp.int32)
    ar = jnp.arange(B, dtype=jnp.int32)
    u_idx = (ar * NU + user_ids) // 2
    i_idx = (ar * NI + item_ids) // 2
    u_par = (user_ids & 1).reshape(B, 1)
    i_par = (item_ids & 1).reshape(B, 1)

    u_flat = user_memory.reshape(B * NU // 2, 2 * D)
    i_flat = item_memory.reshape(B * NI // 2, 2 * D)

    user_emb2, item_emb2 = _sc_gather(u_flat, i_flat, u_idx, i_idx)

    new_user_emb, new_item_emb = _tc_gru(
        user_emb2, item_emb2, u_par, i_par, user_features, item_features,
        w_ih_u, b_ih_u.reshape(1, 3 * H), b_hh_u.reshape(1, 3 * H),
        w_ih_i, b_ih_i.reshape(1, 3 * H), b_hh_i.reshape(1, 3 * H))

    new_user_memory = _tc_copy_scatter(user_memory, new_user_emb, user_ids, NU)
    new_item_memory = _tc_copy_scatter(item_memory, new_item_emb, item_ids, NI)

    return (new_user_emb, new_item_emb, new_user_memory, new_item_memory)


# speedup vs baseline: 1.1914x; 1.1914x over previous
import functools

import jax
import jax.numpy as jnp
from jax import lax
from jax.experimental import pallas as pl
from jax.experimental.pallas import tpu as pltpu
from jax.experimental.pallas import tpu_sc as plsc

B = 512
NU = 1000
NI = 1000
D = 64
FU = 4
FI = 4
IN = D + FU + D + FI
H = D

SC_NC = 2
SC_NS = 16
SC_NW = SC_NC * SC_NS
B_PER_W = B // SC_NW


def _sc_gather(u_flat, i_flat, u_idx, i_idx):
    mesh = plsc.VectorSubcoreMesh(core_axis_name="c", subcore_axis_name="s")

    @functools.partial(
        pl.kernel,
        mesh=mesh,
        out_type=(jax.ShapeDtypeStruct((B, 2 * D), jnp.float32),
                  jax.ShapeDtypeStruct((B, 2 * D), jnp.float32)),
        scratch_types=[
            pltpu.VMEM((B_PER_W,), jnp.int32),
            pltpu.VMEM((B_PER_W, 2 * D), jnp.float32),
            pltpu.SemaphoreType.DMA,
        ],
    )
    def gather_kernel(u_tab, i_tab, u_ix, i_ix, u_out, i_out, idx_v, rows_v, sem):
        wid = lax.axis_index("s") * SC_NC + lax.axis_index("c")
        base = wid * B_PER_W
        pltpu.sync_copy(u_ix.at[pl.ds(base, B_PER_W)], idx_v)
        pltpu.async_copy(u_tab.at[idx_v], rows_v, sem).wait()
        pltpu.sync_copy(rows_v, u_out.at[pl.ds(base, B_PER_W)])
        pltpu.sync_copy(i_ix.at[pl.ds(base, B_PER_W)], idx_v)
        pltpu.async_copy(i_tab.at[idx_v], rows_v, sem).wait()
        pltpu.sync_copy(rows_v, i_out.at[pl.ds(base, B_PER_W)])

    return gather_kernel(u_flat, i_flat, u_idx, i_idx)


def _gru_body(ue2, ie2, up, ip_, uf, if_, wih_u, bih_u, bhh_u,
              wih_i, bih_i, bhh_i, uo, io):
    ue = jnp.where(up[...] == 1, ue2[:, D:], ue2[:, :D])
    ie = jnp.where(ip_[...] == 1, ie2[:, D:], ie2[:, :D])
    for emb, feat, emb2, feat2, wih, bih, bhh, out in (
        (ue, uf, ie, if_, wih_u, bih_u, bhh_u, uo),
        (ie, if_, ue, uf, wih_i, bih_i, bhh_i, io),
    ):
        x = jnp.concatenate(
            [emb, feat[...], emb2, feat2[...]], axis=1)
        gi = lax.dot_general(x, wih[...], (((1,), (1,)), ((), ())),
                             preferred_element_type=jnp.float32)
        gi = gi + bih[...]
        bhh_v = bhh[...]
        r = jax.nn.sigmoid(gi[:, :H] + bhh_v[:, :H])
        z = jax.nn.sigmoid(gi[:, H:2 * H] + bhh_v[:, H:2 * H])
        n = jnp.tanh(gi[:, 2 * H:] + r * bhh_v[:, 2 * H:])
        h = (1.0 - z) * n
        nrm = jnp.sqrt(jnp.sum(h * h, axis=1, keepdims=True))
        out[...] = h / jnp.maximum(nrm, 1e-12)


def _tc_gru(ue2, ie2, up, ip_, uf, if_, wih_u, bih_u, bhh_u,
            wih_i, bih_i, bhh_i):
    return pl.pallas_call(
        _gru_body,
        out_shape=(jax.ShapeDtypeStruct((B, H), jnp.float32),
                   jax.ShapeDtypeStruct((B, H), jnp.float32)),
    )(ue2, ie2, up, ip_, uf, if_, wih_u, bih_u, bhh_u, wih_i, bih_i, bhh_i)


SCATTER_BB = 8


def _scatter_body(mem_ref, emb2_ref, ids_ref, out_ref):
    out_ref[...] = mem_ref[...]
    b0 = pl.program_id(0) * SCATTER_BB
    lane = lax.broadcasted_iota(jnp.int32, (1, 2 * D), 1)
    for i in range(SCATTER_BB):
        idx = ids_ref[b0 + i]
        q = idx // 2
        h0 = (idx & 1) * D
        mask = (lane >= h0) & (lane < h0 + D)
        old = out_ref[i, pl.ds(q, 1), :]
        out_ref[i, pl.ds(q, 1), :] = jnp.where(
            mask, emb2_ref[pl.ds(i, 1), :], old)


def _tc_copy_scatter(mem2, new_emb2, ids, n_rows):
    nh = n_rows // 2
    grid = (B // SCATTER_BB,)
    return pl.pallas_call(
        _scatter_body,
        grid=grid,
        in_specs=[
            pl.BlockSpec((SCATTER_BB, nh, 2 * D), lambda b: (b, 0, 0)),
            pl.BlockSpec((SCATTER_BB, 2 * D), lambda b: (b, 0)),
            pl.BlockSpec(memory_space=pltpu.SMEM),
        ],
        out_specs=pl.BlockSpec((SCATTER_BB, nh, 2 * D), lambda b: (b, 0, 0)),
        out_shape=jax.ShapeDtypeStruct((B, nh, 2 * D), jnp.float32),
    )(mem2, new_emb2, ids)


def kernel(user_ids, item_ids, user_features, item_features,
           user_memory, item_memory,
           w_ih_u, w_hh_u, b_ih_u, b_hh_u,
           w_ih_i, w_hh_i, b_ih_i, b_hh_i):
    user_ids = user_ids.astype(jnp.int32)
    item_ids = item_ids.astype(jnp.int32)
    ar = jnp.arange(B, dtype=jnp.int32)
    u_idx = (ar * NU + user_ids) // 2
    i_idx = (ar * NI + item_ids) // 2
    u_par = (user_ids & 1).reshape(B, 1)
    i_par = (item_ids & 1).reshape(B, 1)

    u_flat = user_memory.reshape(B * NU // 2, 2 * D)
    i_flat = item_memory.reshape(B * NI // 2, 2 * D)

    user_emb2, item_emb2 = _sc_gather(u_flat, i_flat, u_idx, i_idx)

    new_user_emb, new_item_emb = _tc_gru(
        user_emb2, item_emb2, u_par, i_par, user_features, item_features,
        w_ih_u, b_ih_u.reshape(1, 3 * H), b_hh_u.reshape(1, 3 * H),
        w_ih_i, b_ih_i.reshape(1, 3 * H), b_hh_i.reshape(1, 3 * H))

    new_user_emb2 = jnp.concatenate([new_user_emb, new_user_emb], axis=1)
    new_item_emb2 = jnp.concatenate([new_item_emb, new_item_emb], axis=1)
    new_user_memory = _tc_copy_scatter(
        user_memory.reshape(B, NU // 2, 2 * D), new_user_emb2, user_ids, NU)
    new_item_memory = _tc_copy_scatter(
        item_memory.reshape(B, NI // 2, 2 * D), new_item_emb2, item_ids, NI)

    return (new_user_emb,
            new_item_emb,
            new_user_memory.reshape(B, NU, D),
            new_item_memory.reshape(B, NI, D))


# speedup vs baseline: 3.7969x; 3.1869x over previous
import jax
import jax.numpy as jnp
from jax import lax
from jax.experimental import pallas as pl
from jax.experimental.pallas import tpu as pltpu

B = 512
NU = 1000
NI = 1000
D = 64
H = D

RB = 8


def _gather_body(ids_u_ref, ids_i_ref, tu_ref, ti_ref, uo_ref, io_ref,
                 acc_u, acc_i):
    step = pl.program_id(0)
    nsteps = pl.num_programs(0)

    @pl.when(step == 0)
    def _():
        acc_u[...] = jnp.zeros_like(acc_u)
        acc_i[...] = jnp.zeros_like(acc_i)

    ids_u = ids_u_ref[...]
    ids_i = ids_i_ref[...]
    r0 = step * RB
    for rr in range(RB):
        acc_u[...] = jnp.where(ids_u == r0 + rr, tu_ref[rr], acc_u[...])
        acc_i[...] = jnp.where(ids_i == r0 + rr, ti_ref[rr], acc_i[...])

    @pl.when(step == nsteps - 1)
    def _():
        uo_ref[...] = acc_u[...]
        io_ref[...] = acc_i[...]


def _gather(t_u, t_i, ids_u, ids_i):
    return pl.pallas_call(
        _gather_body,
        grid=(NU // RB,),
        in_specs=[
            pl.BlockSpec((1, B), lambda i: (0, 0)),
            pl.BlockSpec((1, B), lambda i: (0, 0)),
            pl.BlockSpec((RB, D, B), lambda i: (i, 0, 0)),
            pl.BlockSpec((RB, D, B), lambda i: (i, 0, 0)),
        ],
        out_specs=[
            pl.BlockSpec((D, B), lambda i: (0, 0)),
            pl.BlockSpec((D, B), lambda i: (0, 0)),
        ],
        out_shape=[
            jax.ShapeDtypeStruct((D, B), jnp.float32),
            jax.ShapeDtypeStruct((D, B), jnp.float32),
        ],
        scratch_shapes=[
            pltpu.VMEM((D, B), jnp.float32),
            pltpu.VMEM((D, B), jnp.float32),
        ],
    )(ids_u, ids_i, t_u, t_i)


def _gru_body(ue, ie, uf, if_, wih_u, bih_u, bhh_u, wih_i, bih_i, bhh_i,
              uo, io):
    ue_v, ie_v, uf_v, if_v = ue[...], ie[...], uf[...], if_[...]
    for emb, feat, emb2, feat2, wih, bih, bhh, out in (
        (ue_v, uf_v, ie_v, if_v, wih_u, bih_u, bhh_u, uo),
        (ie_v, if_v, ue_v, uf_v, wih_i, bih_i, bhh_i, io),
    ):
        x = jnp.concatenate([emb, feat, emb2, feat2], axis=0)
        gi = lax.dot_general(wih[...], x, (((1,), (0,)), ((), ())),
                             preferred_element_type=jnp.float32)
        gi = gi + bih[...]
        bhh_v = bhh[...]
        r = jax.nn.sigmoid(gi[:H] + bhh_v[:H])
        z = jax.nn.sigmoid(gi[H:2 * H] + bhh_v[H:2 * H])
        n = jnp.tanh(gi[2 * H:] + r * bhh_v[2 * H:])
        h = (1.0 - z) * n
        nrm = jnp.sqrt(jnp.sum(h * h, axis=0, keepdims=True))
        out[...] = h / jnp.maximum(nrm, 1e-12)


def _gru(ue_t, ie_t, uf_t, if_t, wih_u, bih_u, bhh_u, wih_i, bih_i, bhh_i):
    return pl.pallas_call(
        _gru_body,
        out_shape=(jax.ShapeDtypeStruct((H, B), jnp.float32),
                   jax.ShapeDtypeStruct((H, B), jnp.float32)),
    )(ue_t, ie_t, uf_t, if_t, wih_u, bih_u, bhh_u, wih_i, bih_i, bhh_i)


def _scatter_body(ids_ref, emb_ref, t_ref, out_ref):
    r0 = pl.program_id(0) * RB
    ids = ids_ref[...]
    emb = emb_ref[...]
    for rr in range(RB):
        out_ref[rr] = jnp.where(ids == r0 + rr, emb, t_ref[rr])


def _copy_scatter(t_mem, emb_t, ids, n_rows):
    return pl.pallas_call(
        _scatter_body,
        grid=(n_rows // RB,),
        in_specs=[
            pl.BlockSpec((1, B), lambda i: (0, 0)),
            pl.BlockSpec((D, B), lambda i: (0, 0)),
            pl.BlockSpec((RB, D, B), lambda i: (i, 0, 0)),
        ],
        out_specs=pl.BlockSpec((RB, D, B), lambda i: (i, 0, 0)),
        out_shape=jax.ShapeDtypeStruct((n_rows, D, B), jnp.float32),
    )(ids, emb_t, t_mem)


def kernel(user_ids, item_ids, user_features, item_features,
           user_memory, item_memory,
           w_ih_u, w_hh_u, b_ih_u, b_hh_u,
           w_ih_i, w_hh_i, b_ih_i, b_hh_i):
    ids_u = user_ids.astype(jnp.int32).reshape(1, B)
    ids_i = item_ids.astype(jnp.int32).reshape(1, B)

    t_u = jnp.transpose(user_memory, (1, 2, 0))
    t_i = jnp.transpose(item_memory, (1, 2, 0))
    uf_t = jnp.transpose(user_features)
    if_t = jnp.transpose(item_features)

    ue_t, ie_t = _gather(t_u, t_i, ids_u, ids_i)

    new_ue_t, new_ie_t = _gru(
        ue_t, ie_t, uf_t, if_t,
        w_ih_u, b_ih_u.reshape(3 * H, 1), b_hh_u.reshape(3 * H, 1),
        w_ih_i, b_ih_i.reshape(3 * H, 1), b_hh_i.reshape(3 * H, 1))

    new_t_u = _copy_scatter(t_u, new_ue_t, ids_u, NU)
    new_t_i = _copy_scatter(t_i, new_ie_t, ids_i, NI)

    return (jnp.transpose(new_ue_t),
            jnp.transpose(new_ie_t),
            jnp.transpose(new_t_u, (2, 0, 1)),
            jnp.transpose(new_t_i, (2, 0, 1)))


# speedup vs baseline: 4.7074x; 1.2398x over previous
import jax
import jax.numpy as jnp
from jax import lax
from jax.experimental import pallas as pl
from jax.experimental.pallas import tpu as pltpu

B = 512
NU = 1000
NI = 1000
D = 64
H = D
FU = 4
FI = 4
IN = D + FU + D + FI

RB = 8


def _gru_compute(ue_v, ie_v, uf_v, if_v, wih, bih, bhh):
    x = jnp.concatenate([ue_v, uf_v, ie_v, if_v], axis=0)
    gi = lax.dot_general(wih, x, (((1,), (0,)), ((), ())),
                         preferred_element_type=jnp.float32)
    gi = gi + bih
    r = jax.nn.sigmoid(gi[:H] + bhh[:H])
    z = jax.nn.sigmoid(gi[H:2 * H] + bhh[H:2 * H])
    n = jnp.tanh(gi[2 * H:] + r * bhh[2 * H:])
    h = (1.0 - z) * n
    nrm = jnp.sqrt(jnp.sum(h * h, axis=0, keepdims=True))
    return h / jnp.maximum(nrm, 1e-12)


def _gather_gru_body(ids_u_ref, ids_i_ref, tu_ref, ti_ref,
                     wih_u, bih_u, bhh_u, wih_i, bih_i, bhh_i,
                     uf_ref, if_ref, uo_ref, io_ref, acc_u, acc_i):
    step = pl.program_id(0)
    nsteps = pl.num_programs(0)

    @pl.when(step == 0)
    def _():
        acc_u[...] = jnp.zeros_like(acc_u)
        acc_i[...] = jnp.zeros_like(acc_i)

    ids_u = ids_u_ref[...]
    ids_i = ids_i_ref[...]
    r0 = step * RB
    for rr in range(RB):
        acc_u[...] = jnp.where(ids_u == r0 + rr, tu_ref[rr], acc_u[...])
        acc_i[...] = jnp.where(ids_i == r0 + rr, ti_ref[rr], acc_i[...])

    @pl.when(step == nsteps - 1)
    def _():
        ue_v, ie_v = acc_u[...], acc_i[...]
        uf_v, if_v = uf_ref[...], if_ref[...]
        uo_ref[...] = _gru_compute(ue_v, ie_v, uf_v, if_v,
                                   wih_u[...], bih_u[...], bhh_u[...])
        io_ref[...] = _gru_compute(ie_v, ue_v, if_v, uf_v,
                                   wih_i[...], bih_i[...], bhh_i[...])


def _gather_gru(t_u, t_i, ids_u, ids_i, uf_t, if_t,
                wih_u, bih_u, bhh_u, wih_i, bih_i, bhh_i):
    c0 = lambda i: (0, 0)
    return pl.pallas_call(
        _gather_gru_body,
        grid=(NU // RB,),
        in_specs=[
            pl.BlockSpec((1, B), c0),
            pl.BlockSpec((1, B), c0),
            pl.BlockSpec((RB, D, B), lambda i: (i, 0, 0)),
            pl.BlockSpec((RB, D, B), lambda i: (i, 0, 0)),
            pl.BlockSpec((3 * H, IN), c0),
            pl.BlockSpec((3 * H, 1), c0),
            pl.BlockSpec((3 * H, 1), c0),
            pl.BlockSpec((3 * H, IN), c0),
            pl.BlockSpec((3 * H, 1), c0),
            pl.BlockSpec((3 * H, 1), c0),
            pl.BlockSpec((FU, B), c0),
            pl.BlockSpec((FI, B), c0),
        ],
        out_specs=[
            pl.BlockSpec((D, B), c0),
            pl.BlockSpec((D, B), c0),
        ],
        out_shape=[
            jax.ShapeDtypeStruct((D, B), jnp.float32),
            jax.ShapeDtypeStruct((D, B), jnp.float32),
        ],
        scratch_shapes=[
            pltpu.VMEM((D, B), jnp.float32),
            pltpu.VMEM((D, B), jnp.float32),
        ],
    )(ids_u, ids_i, t_u, t_i, wih_u, bih_u, bhh_u, wih_i, bih_i, bhh_i,
      uf_t, if_t)


def _scatter_body(ids_u_ref, ids_i_ref, emb_u_ref, emb_i_ref,
                  tu_ref, ti_ref, uo_ref, io_ref):
    r0 = pl.program_id(0) * RB
    ids_u = ids_u_ref[...]
    ids_i = ids_i_ref[...]
    emb_u = emb_u_ref[...]
    emb_i = emb_i_ref[...]
    for rr in range(RB):
        uo_ref[rr] = jnp.where(ids_u == r0 + rr, emb_u, tu_ref[rr])
        io_ref[rr] = jnp.where(ids_i == r0 + rr, emb_i, ti_ref[rr])


def _copy_scatter(t_u, t_i, emb_u_t, emb_i_t, ids_u, ids_i):
    c0 = lambda i: (0, 0)
    blk = lambda i: (i, 0, 0)
    return pl.pallas_call(
        _scatter_body,
        grid=(NU // RB,),
        in_specs=[
            pl.BlockSpec((1, B), c0),
            pl.BlockSpec((1, B), c0),
            pl.BlockSpec((D, B), c0),
            pl.BlockSpec((D, B), c0),
            pl.BlockSpec((RB, D, B), blk),
            pl.BlockSpec((RB, D, B), blk),
        ],
        out_specs=[
            pl.BlockSpec((RB, D, B), blk),
            pl.BlockSpec((RB, D, B), blk),
        ],
        out_shape=[
            jax.ShapeDtypeStruct((NU, D, B), jnp.float32),
            jax.ShapeDtypeStruct((NI, D, B), jnp.float32),
        ],
    )(ids_u, ids_i, emb_u_t, emb_i_t, t_u, t_i)


def kernel(user_ids, item_ids, user_features, item_features,
           user_memory, item_memory,
           w_ih_u, w_hh_u, b_ih_u, b_hh_u,
           w_ih_i, w_hh_i, b_ih_i, b_hh_i):
    ids_u = user_ids.astype(jnp.int32).reshape(1, B)
    ids_i = item_ids.astype(jnp.int32).reshape(1, B)

    t_u = jnp.transpose(user_memory, (1, 2, 0))
    t_i = jnp.transpose(item_memory, (1, 2, 0))
    uf_t = jnp.transpose(user_features)
    if_t = jnp.transpose(item_features)

    new_ue_t, new_ie_t = _gather_gru(
        t_u, t_i, ids_u, ids_i, uf_t, if_t,
        w_ih_u, b_ih_u.reshape(3 * H, 1), b_hh_u.reshape(3 * H, 1),
        w_ih_i, b_ih_i.reshape(3 * H, 1), b_hh_i.reshape(3 * H, 1))

    new_t_u, new_t_i = _copy_scatter(t_u, t_i, new_ue_t, new_ie_t,
                                     ids_u, ids_i)

    return (jnp.transpose(new_ue_t),
            jnp.transpose(new_ie_t),
            jnp.transpose(new_t_u, (2, 0, 1)),
            jnp.transpose(new_t_i, (2, 0, 1)))


# speedup vs baseline: 4.7419x; 1.0073x over previous
import jax
import jax.numpy as jnp
from jax import lax
from jax.experimental import pallas as pl
from jax.experimental.pallas import tpu as pltpu

B = 512
NU = 1000
NI = 1000
D = 64
H = D
FU = 4
FI = 4
IN = D + FU + D + FI

RB = 8


def _gru_compute(ue_v, ie_v, uf_v, if_v, wih, bih, bhh):
    x = jnp.concatenate([ue_v, uf_v, ie_v, if_v], axis=0)
    gi = lax.dot_general(wih, x, (((1,), (0,)), ((), ())),
                         preferred_element_type=jnp.float32)
    gi = gi + bih
    r = jax.nn.sigmoid(gi[:H] + bhh[:H])
    z = jax.nn.sigmoid(gi[H:2 * H] + bhh[H:2 * H])
    n = jnp.tanh(gi[2 * H:] + r * bhh[2 * H:])
    h = (1.0 - z) * n
    nrm = jnp.sqrt(jnp.sum(h * h, axis=0, keepdims=True))
    return h / jnp.maximum(nrm, 1e-12)


def _gather_gru_body(ids_u_ref, ids_i_ref, tu_ref, ti_ref,
                     wih_u, bih_u, bhh_u, wih_i, bih_i, bhh_i,
                     uf_ref, if_ref, uo_ref, io_ref, acc_u, acc_i):
    step = pl.program_id(0)
    nsteps = pl.num_programs(0)

    @pl.when(step == 0)
    def _():
        acc_u[...] = jnp.zeros_like(acc_u)
        acc_i[...] = jnp.zeros_like(acc_i)

    ids_u = ids_u_ref[...]
    ids_i = ids_i_ref[...]
    r0 = step * RB
    for ids, t_ref, acc in ((ids_u, tu_ref, acc_u), (ids_i, ti_ref, acc_i)):
        masks = [ids == r0 + rr for rr in range(RB)]
        cand = t_ref[RB - 1]
        for rr in range(RB - 2, -1, -1):
            cand = jnp.where(masks[rr], t_ref[rr], cand)
        m_any = masks[0]
        for rr in range(1, RB):
            m_any = m_any | masks[rr]
        acc[...] = jnp.where(m_any, cand, acc[...])

    @pl.when(step == nsteps - 1)
    def _():
        ue_v, ie_v = acc_u[...], acc_i[...]
        uf_v, if_v = uf_ref[...], if_ref[...]
        uo_ref[...] = _gru_compute(ue_v, ie_v, uf_v, if_v,
                                   wih_u[...], bih_u[...], bhh_u[...])
        io_ref[...] = _gru_compute(ie_v, ue_v, if_v, uf_v,
                                   wih_i[...], bih_i[...], bhh_i[...])


def _gather_gru(t_u, t_i, ids_u, ids_i, uf_t, if_t,
                wih_u, bih_u, bhh_u, wih_i, bih_i, bhh_i):
    c0 = lambda i: (0, 0)
    return pl.pallas_call(
        _gather_gru_body,
        grid=(NU // RB,),
        in_specs=[
            pl.BlockSpec((1, B), c0),
            pl.BlockSpec((1, B), c0),
            pl.BlockSpec((RB, D, B), lambda i: (i, 0, 0)),
            pl.BlockSpec((RB, D, B), lambda i: (i, 0, 0)),
            pl.BlockSpec((3 * H, IN), c0),
            pl.BlockSpec((3 * H, 1), c0),
            pl.BlockSpec((3 * H, 1), c0),
            pl.BlockSpec((3 * H, IN), c0),
            pl.BlockSpec((3 * H, 1), c0),
            pl.BlockSpec((3 * H, 1), c0),
            pl.BlockSpec((FU, B), c0),
            pl.BlockSpec((FI, B), c0),
        ],
        out_specs=[
            pl.BlockSpec((D, B), c0),
            pl.BlockSpec((D, B), c0),
        ],
        out_shape=[
            jax.ShapeDtypeStruct((D, B), jnp.float32),
            jax.ShapeDtypeStruct((D, B), jnp.float32),
        ],
        scratch_shapes=[
            pltpu.VMEM((D, B), jnp.float32),
            pltpu.VMEM((D, B), jnp.float32),
        ],
    )(ids_u, ids_i, t_u, t_i, wih_u, bih_u, bhh_u, wih_i, bih_i, bhh_i,
      uf_t, if_t)


def _scatter_body(ids_u_ref, ids_i_ref, emb_u_ref, emb_i_ref,
                  tu_ref, ti_ref, uo_ref, io_ref):
    r0 = pl.program_id(0) * RB
    ids_u = ids_u_ref[...]
    ids_i = ids_i_ref[...]
    emb_u = emb_u_ref[...]
    emb_i = emb_i_ref[...]
    for rr in range(RB):
        uo_ref[rr] = jnp.where(ids_u == r0 + rr, emb_u, tu_ref[rr])
        io_ref[rr] = jnp.where(ids_i == r0 + rr, emb_i, ti_ref[rr])


def _copy_scatter(t_u, t_i, emb_u_t, emb_i_t, ids_u, ids_i):
    c0 = lambda i: (0, 0)
    blk = lambda i: (i, 0, 0)
    return pl.pallas_call(
        _scatter_body,
        grid=(NU // RB,),
        in_specs=[
            pl.BlockSpec((1, B), c0),
            pl.BlockSpec((1, B), c0),
            pl.BlockSpec((D, B), c0),
            pl.BlockSpec((D, B), c0),
            pl.BlockSpec((RB, D, B), blk),
            pl.BlockSpec((RB, D, B), blk),
        ],
        out_specs=[
            pl.BlockSpec((RB, D, B), blk),
            pl.BlockSpec((RB, D, B), blk),
        ],
        out_shape=[
            jax.ShapeDtypeStruct((NU, D, B), jnp.float32),
            jax.ShapeDtypeStruct((NI, D, B), jnp.float32),
        ],
    )(ids_u, ids_i, emb_u_t, emb_i_t, t_u, t_i)


def kernel(user_ids, item_ids, user_features, item_features,
           user_memory, item_memory,
           w_ih_u, w_hh_u, b_ih_u, b_hh_u,
           w_ih_i, w_hh_i, b_ih_i, b_hh_i):
    ids_u = user_ids.astype(jnp.int32).reshape(1, B)
    ids_i = item_ids.astype(jnp.int32).reshape(1, B)

    t_u = jnp.transpose(user_memory, (1, 2, 0))
    t_i = jnp.transpose(item_memory, (1, 2, 0))
    uf_t = jnp.transpose(user_features)
    if_t = jnp.transpose(item_features)

    new_ue_t, new_ie_t = _gather_gru(
        t_u, t_i, ids_u, ids_i, uf_t, if_t,
        w_ih_u, b_ih_u.reshape(3 * H, 1), b_hh_u.reshape(3 * H, 1),
        w_ih_i, b_ih_i.reshape(3 * H, 1), b_hh_i.reshape(3 * H, 1))

    new_t_u, new_t_i = _copy_scatter(t_u, t_i, new_ue_t, new_ie_t,
                                     ids_u, ids_i)

    return (jnp.transpose(new_ue_t),
            jnp.transpose(new_ie_t),
            jnp.transpose(new_t_u, (2, 0, 1)),
            jnp.transpose(new_t_i, (2, 0, 1)))


# speedup vs baseline: 5.6911x; 1.2002x over previous
import jax
import jax.numpy as jnp
from jax import lax
from jax.experimental import pallas as pl
from jax.experimental.pallas import tpu as pltpu

B = 512
NU = 1000
NI = 1000
D = 64
H = D
FU = 4
FI = 4
IN = D + FU + D + FI

RB = 20


def _gru_compute(ue_v, ie_v, uf_v, if_v, wih, bih, bhh):
    x = jnp.concatenate([ue_v, uf_v, ie_v, if_v], axis=0)
    gi = lax.dot_general(wih, x, (((1,), (0,)), ((), ())),
                         preferred_element_type=jnp.float32)
    gi = gi + bih
    r = jax.nn.sigmoid(gi[:H] + bhh[:H])
    z = jax.nn.sigmoid(gi[H:2 * H] + bhh[H:2 * H])
    n = jnp.tanh(gi[2 * H:] + r * bhh[2 * H:])
    h = (1.0 - z) * n
    nrm = jnp.sqrt(jnp.sum(h * h, axis=0, keepdims=True))
    return h / jnp.maximum(nrm, 1e-12)


def _gather_gru_body(ids_u_ref, ids_i_ref, tu_ref, ti_ref,
                     wih_u, bih_u, bhh_u, wih_i, bih_i, bhh_i,
                     uf_ref, if_ref, uo_ref, io_ref, acc_u, acc_i):
    step = pl.program_id(0)
    nsteps = pl.num_programs(0)

    @pl.when(step == 0)
    def _():
        acc_u[...] = jnp.zeros_like(acc_u)
        acc_i[...] = jnp.zeros_like(acc_i)

    ids_u = ids_u_ref[...]
    ids_i = ids_i_ref[...]
    r0 = step * RB
    for ids, t_ref, acc in ((ids_u, tu_ref, acc_u), (ids_i, ti_ref, acc_i)):
        masks = [ids == r0 + rr for rr in range(RB)]
        cand = t_ref[RB - 1]
        for rr in range(RB - 2, -1, -1):
            cand = jnp.where(masks[rr], t_ref[rr], cand)
        m_any = masks[0]
        for rr in range(1, RB):
            m_any = m_any | masks[rr]
        acc[...] = jnp.where(m_any, cand, acc[...])

    @pl.when(step == nsteps - 1)
    def _():
        ue_v, ie_v = acc_u[...], acc_i[...]
        uf_v, if_v = uf_ref[...], if_ref[...]
        uo_ref[...] = _gru_compute(ue_v, ie_v, uf_v, if_v,
                                   wih_u[...], bih_u[...], bhh_u[...])
        io_ref[...] = _gru_compute(ie_v, ue_v, if_v, uf_v,
                                   wih_i[...], bih_i[...], bhh_i[...])


def _gather_gru(t_u, t_i, ids_u, ids_i, uf_t, if_t,
                wih_u, bih_u, bhh_u, wih_i, bih_i, bhh_i):
    c0 = lambda i: (0, 0)
    return pl.pallas_call(
        _gather_gru_body,
        grid=(NU // RB,),
        in_specs=[
            pl.BlockSpec((1, B), c0),
            pl.BlockSpec((1, B), c0),
            pl.BlockSpec((RB, D, B), lambda i: (i, 0, 0)),
            pl.BlockSpec((RB, D, B), lambda i: (i, 0, 0)),
            pl.BlockSpec((3 * H, IN), c0),
            pl.BlockSpec((3 * H, 1), c0),
            pl.BlockSpec((3 * H, 1), c0),
            pl.BlockSpec((3 * H, IN), c0),
            pl.BlockSpec((3 * H, 1), c0),
            pl.BlockSpec((3 * H, 1), c0),
            pl.BlockSpec((FU, B), c0),
            pl.BlockSpec((FI, B), c0),
        ],
        out_specs=[
            pl.BlockSpec((D, B), c0),
            pl.BlockSpec((D, B), c0),
        ],
        out_shape=[
            jax.ShapeDtypeStruct((D, B), jnp.float32),
            jax.ShapeDtypeStruct((D, B), jnp.float32),
        ],
        scratch_shapes=[
            pltpu.VMEM((D, B), jnp.float32),
            pltpu.VMEM((D, B), jnp.float32),
        ],
    )(ids_u, ids_i, t_u, t_i, wih_u, bih_u, bhh_u, wih_i, bih_i, bhh_i,
      uf_t, if_t)


def _scatter_body(ids_u_ref, ids_i_ref, emb_u_ref, emb_i_ref,
                  tu_ref, ti_ref, uo_ref, io_ref):
    r0 = pl.program_id(0) * RB
    ids_u = ids_u_ref[...]
    ids_i = ids_i_ref[...]
    emb_u = emb_u_ref[...]
    emb_i = emb_i_ref[...]
    for rr in range(RB):
        uo_ref[rr] = jnp.where(ids_u == r0 + rr, emb_u, tu_ref[rr])
        io_ref[rr] = jnp.where(ids_i == r0 + rr, emb_i, ti_ref[rr])


def _copy_scatter(t_u, t_i, emb_u_t, emb_i_t, ids_u, ids_i):
    c0 = lambda i: (0, 0)
    blk = lambda i: (i, 0, 0)
    return pl.pallas_call(
        _scatter_body,
        grid=(NU // RB,),
        in_specs=[
            pl.BlockSpec((1, B), c0),
            pl.BlockSpec((1, B), c0),
            pl.BlockSpec((D, B), c0),
            pl.BlockSpec((D, B), c0),
            pl.BlockSpec((RB, D, B), blk),
            pl.BlockSpec((RB, D, B), blk),
        ],
        out_specs=[
            pl.BlockSpec((RB, D, B), blk),
            pl.BlockSpec((RB, D, B), blk),
        ],
        out_shape=[
            jax.ShapeDtypeStruct((NU, D, B), jnp.float32),
            jax.ShapeDtypeStruct((NI, D, B), jnp.float32),
        ],
    )(ids_u, ids_i, emb_u_t, emb_i_t, t_u, t_i)


def kernel(user_ids, item_ids, user_features, item_features,
           user_memory, item_memory,
           w_ih_u, w_hh_u, b_ih_u, b_hh_u,
           w_ih_i, w_hh_i, b_ih_i, b_hh_i):
    ids_u = user_ids.astype(jnp.int32).reshape(1, B)
    ids_i = item_ids.astype(jnp.int32).reshape(1, B)

    t_u = jnp.transpose(user_memory, (1, 2, 0))
    t_i = jnp.transpose(item_memory, (1, 2, 0))
    uf_t = jnp.transpose(user_features)
    if_t = jnp.transpose(item_features)

    new_ue_t, new_ie_t = _gather_gru(
        t_u, t_i, ids_u, ids_i, uf_t, if_t,
        w_ih_u, b_ih_u.reshape(3 * H, 1), b_hh_u.reshape(3 * H, 1),
        w_ih_i, b_ih_i.reshape(3 * H, 1), b_hh_i.reshape(3 * H, 1))

    new_t_u, new_t_i = _copy_scatter(t_u, t_i, new_ue_t, new_ie_t,
                                     ids_u, ids_i)

    return (jnp.transpose(new_ue_t),
            jnp.transpose(new_ie_t),
            jnp.transpose(new_t_u, (2, 0, 1)),
            jnp.transpose(new_t_i, (2, 0, 1)))


# speedup vs baseline: 5.8284x; 1.0241x over previous
import jax
import jax.numpy as jnp
from jax import lax
from jax.experimental import pallas as pl
from jax.experimental.pallas import tpu as pltpu

B = 512
NU = 1000
NI = 1000
D = 64
H = D
FU = 4
FI = 4
IN = D + FU + D + FI

RB = 40


def _gru_compute(ue_v, ie_v, uf_v, if_v, wih, bih, bhh):
    x = jnp.concatenate([ue_v, uf_v, ie_v, if_v], axis=0)
    gi = lax.dot_general(wih, x, (((1,), (0,)), ((), ())),
                         preferred_element_type=jnp.float32)
    gi = gi + bih
    r = jax.nn.sigmoid(gi[:H] + bhh[:H])
    z = jax.nn.sigmoid(gi[H:2 * H] + bhh[H:2 * H])
    n = jnp.tanh(gi[2 * H:] + r * bhh[2 * H:])
    h = (1.0 - z) * n
    nrm = jnp.sqrt(jnp.sum(h * h, axis=0, keepdims=True))
    return h / jnp.maximum(nrm, 1e-12)


def _gather_gru_body(ids_u_ref, ids_i_ref, tu_ref, ti_ref,
                     wih_u, bih_u, bhh_u, wih_i, bih_i, bhh_i,
                     uf_ref, if_ref, uo_ref, io_ref, acc_u, acc_i):
    step = pl.program_id(0)
    nsteps = pl.num_programs(0)

    @pl.when(step == 0)
    def _():
        acc_u[...] = jnp.zeros_like(acc_u)
        acc_i[...] = jnp.zeros_like(acc_i)

    ids_u = ids_u_ref[...]
    ids_i = ids_i_ref[...]
    r0 = step * RB
    for ids, t_ref, acc in ((ids_u, tu_ref, acc_u), (ids_i, ti_ref, acc_i)):
        masks = [ids == r0 + rr for rr in range(RB)]
        cand = t_ref[RB - 1]
        for rr in range(RB - 2, -1, -1):
            cand = jnp.where(masks[rr], t_ref[rr], cand)
        m_any = masks[0]
        for rr in range(1, RB):
            m_any = m_any | masks[rr]
        acc[...] = jnp.where(m_any, cand, acc[...])

    @pl.when(step == nsteps - 1)
    def _():
        ue_v, ie_v = acc_u[...], acc_i[...]
        uf_v, if_v = uf_ref[...], if_ref[...]
        uo_ref[...] = _gru_compute(ue_v, ie_v, uf_v, if_v,
                                   wih_u[...], bih_u[...], bhh_u[...])
        io_ref[...] = _gru_compute(ie_v, ue_v, if_v, uf_v,
                                   wih_i[...], bih_i[...], bhh_i[...])


def _gather_gru(t_u, t_i, ids_u, ids_i, uf_t, if_t,
                wih_u, bih_u, bhh_u, wih_i, bih_i, bhh_i):
    c0 = lambda i: (0, 0)
    return pl.pallas_call(
        _gather_gru_body,
        grid=(NU // RB,),
        in_specs=[
            pl.BlockSpec((1, B), c0),
            pl.BlockSpec((1, B), c0),
            pl.BlockSpec((RB, D, B), lambda i: (i, 0, 0)),
            pl.BlockSpec((RB, D, B), lambda i: (i, 0, 0)),
            pl.BlockSpec((3 * H, IN), c0),
            pl.BlockSpec((3 * H, 1), c0),
            pl.BlockSpec((3 * H, 1), c0),
            pl.BlockSpec((3 * H, IN), c0),
            pl.BlockSpec((3 * H, 1), c0),
            pl.BlockSpec((3 * H, 1), c0),
            pl.BlockSpec((FU, B), c0),
            pl.BlockSpec((FI, B), c0),
        ],
        out_specs=[
            pl.BlockSpec((D, B), c0),
            pl.BlockSpec((D, B), c0),
        ],
        out_shape=[
            jax.ShapeDtypeStruct((D, B), jnp.float32),
            jax.ShapeDtypeStruct((D, B), jnp.float32),
        ],
        scratch_shapes=[
            pltpu.VMEM((D, B), jnp.float32),
            pltpu.VMEM((D, B), jnp.float32),
        ],
    )(ids_u, ids_i, t_u, t_i, wih_u, bih_u, bhh_u, wih_i, bih_i, bhh_i,
      uf_t, if_t)


def _scatter_body(ids_u_ref, ids_i_ref, emb_u_ref, emb_i_ref,
                  tu_ref, ti_ref, uo_ref, io_ref):
    r0 = pl.program_id(0) * RB
    ids_u = ids_u_ref[...]
    ids_i = ids_i_ref[...]
    emb_u = emb_u_ref[...]
    emb_i = emb_i_ref[...]
    for rr in range(RB):
        uo_ref[rr] = jnp.where(ids_u == r0 + rr, emb_u, tu_ref[rr])
        io_ref[rr] = jnp.where(ids_i == r0 + rr, emb_i, ti_ref[rr])


def _copy_scatter(t_u, t_i, emb_u_t, emb_i_t, ids_u, ids_i):
    c0 = lambda i: (0, 0)
    blk = lambda i: (i, 0, 0)
    return pl.pallas_call(
        _scatter_body,
        grid=(NU // RB,),
        in_specs=[
            pl.BlockSpec((1, B), c0),
            pl.BlockSpec((1, B), c0),
            pl.BlockSpec((D, B), c0),
            pl.BlockSpec((D, B), c0),
            pl.BlockSpec((RB, D, B), blk),
            pl.BlockSpec((RB, D, B), blk),
        ],
        out_specs=[
            pl.BlockSpec((RB, D, B), blk),
            pl.BlockSpec((RB, D, B), blk),
        ],
        out_shape=[
            jax.ShapeDtypeStruct((NU, D, B), jnp.float32),
            jax.ShapeDtypeStruct((NI, D, B), jnp.float32),
        ],
    )(ids_u, ids_i, emb_u_t, emb_i_t, t_u, t_i)


def kernel(user_ids, item_ids, user_features, item_features,
           user_memory, item_memory,
           w_ih_u, w_hh_u, b_ih_u, b_hh_u,
           w_ih_i, w_hh_i, b_ih_i, b_hh_i):
    ids_u = user_ids.astype(jnp.int32).reshape(1, B)
    ids_i = item_ids.astype(jnp.int32).reshape(1, B)

    t_u = jnp.transpose(user_memory, (1, 2, 0))
    t_i = jnp.transpose(item_memory, (1, 2, 0))
    uf_t = jnp.transpose(user_features)
    if_t = jnp.transpose(item_features)

    new_ue_t, new_ie_t = _gather_gru(
        t_u, t_i, ids_u, ids_i, uf_t, if_t,
        w_ih_u, b_ih_u.reshape(3 * H, 1), b_hh_u.reshape(3 * H, 1),
        w_ih_i, b_ih_i.reshape(3 * H, 1), b_hh_i.reshape(3 * H, 1))

    new_t_u, new_t_i = _copy_scatter(t_u, t_i, new_ue_t, new_ie_t,
                                     ids_u, ids_i)

    return (jnp.transpose(new_ue_t),
            jnp.transpose(new_ie_t),
            jnp.transpose(new_t_u, (2, 0, 1)),
            jnp.transpose(new_t_i, (2, 0, 1)))
